# baseline (speedup 1.0000x reference)
import functools

import jax
import jax.numpy as jnp
from jax import lax
from jax.experimental import pallas as pl
from jax.experimental.pallas import tpu as pltpu
from jax.experimental.pallas import tpu_sc as plsc

_BR = 50
_K = 16


def _mlp_body(ff_ref, ev3_ref, p_ref, w0_ref, b0_ref, w1c_ref,
              b1_ref, vx_ref, vy_ref, vz_ref):
    x = ff_ref[...]
    ht = jax.lax.dot_general(w0_ref[...], x, (((0,), (1,)), ((), ())),
                             preferred_element_type=jnp.float32)
    ht = ht + b0_ref[...].reshape(128, 1)
    ht = ht * (1.0 / (1.0 + jnp.exp(-ht)))
    st = jax.lax.dot_general(w1c_ref[...], ht, (((0,), (0,)), ((), ())),
                             preferred_element_type=jnp.float32)
    s2 = (st + b1_ref[0, 0]).reshape(_BR, 128)
    ev = jnp.dot(ev3_ref[0], p_ref[...], preferred_element_type=jnp.float32)
    vx_ref[0] = s2 * ev[:, 0:128]
    vy_ref[0] = s2 * ev[:, 128:256]
    vz_ref[0] = s2 * ev[:, 256:384]


def _mlp_call(ff, ev3, pmat, w0, b0r, w1c, b1r):
    e, d = ff.shape
    grid = e // (128 * _BR)
    be = _BR * 128
    out_spec = pl.BlockSpec((1, _BR, 128), lambda i: (i, 0, 0))
    out_sds = jax.ShapeDtypeStruct((grid, _BR, 128), jnp.float32)
    return pl.pallas_call(
        _mlp_body,
        grid=(grid,),
        in_specs=[
            pl.BlockSpec((be, d), lambda i: (i, 0)),
            pl.BlockSpec((1, _BR, 384), lambda i: (i, 0, 0)),
            pl.BlockSpec((384, 384), lambda i: (0, 0)),
            pl.BlockSpec((d, d), lambda i: (0, 0)),
            pl.BlockSpec((1, d), lambda i: (0, 0)),
            pl.BlockSpec((d, 1), lambda i: (0, 0)),
            pl.BlockSpec((1, 1), lambda i: (0, 0)),
        ],
        out_specs=[out_spec, out_spec, out_spec],
        out_shape=[out_sds, out_sds, out_sds],
    )(ff, ev3, pmat, w0, b0r, w1c, b1r)


def _make_sc_scatter(rows_pad, n_pad):
    rows_per_tile = rows_pad // 32
    n_batches = rows_per_tile // _K
    per_s = n_pad // 16
    mesh = plsc.VectorSubcoreMesh(core_axis_name="c", subcore_axis_name="s")

    @functools.partial(
        pl.kernel,
        out_type=jax.ShapeDtypeStruct((6, n_pad), jnp.float32),
        mesh=mesh,
        scratch_types=[
            pltpu.VMEM((2, _K, 128), jnp.int32),
            pltpu.VMEM((2, _K, 128), jnp.float32),
            pltpu.VMEM((2, _K, 128), jnp.float32),
            pltpu.VMEM((2, _K, 128), jnp.float32),
            pltpu.VMEM_SHARED((n_pad,), jnp.float32),
            pltpu.VMEM_SHARED((n_pad,), jnp.float32),
            pltpu.VMEM_SHARED((n_pad,), jnp.float32),
            pltpu.SemaphoreType.DMA,
            pltpu.SemaphoreType.DMA,
            pltpu.SemaphoreType.DMA,
            pltpu.SemaphoreType.DMA,
        ],
    )
    def sc_scatter(vx_hbm, vy_hbm, vz_hbm, dst_hbm, zeros_hbm, out_hbm,
                   idx2, vx2, vy2, vz2, acc_x, acc_y, acc_z,
                   sem_st0, sem_st1, sem_sc0, sem_sc1):
        c = lax.axis_index("c")
        s = lax.axis_index("s")
        wid = s * 2 + c
        sl = pl.ds(s * per_s, per_s)
        pltpu.sync_copy(zeros_hbm.at[sl], acc_x.at[sl])
        pltpu.sync_copy(zeros_hbm.at[sl], acc_y.at[sl])
        pltpu.sync_copy(zeros_hbm.at[sl], acc_z.at[sl])
        plsc.subcore_barrier()
        start_row = wid * rows_per_tile
        sem_st = (sem_st0, sem_st1)
        sem_sc = (sem_sc0, sem_sc1)

        def stage(b):
            p = b % 2
            r0 = start_row + b * _K
            sem = sem_st[p]
            return [
                pltpu.async_copy(dst_hbm.at[pl.ds(r0, _K)], idx2.at[p], sem),
                pltpu.async_copy(vx_hbm.at[pl.ds(r0, _K)], vx2.at[p], sem),
                pltpu.async_copy(vy_hbm.at[pl.ds(r0, _K)], vy2.at[p], sem),
                pltpu.async_copy(vz_hbm.at[pl.ds(r0, _K)], vz2.at[p], sem),
            ]

        def fire(b):
            p = b % 2
            sem = sem_sc[p]
            descs = []
            for j in range(_K):
                row_idx = idx2.at[p, j]
                descs.append(pltpu.async_copy(
                    vx2.at[p, j], acc_x.at[row_idx], sem, add=True))
                descs.append(pltpu.async_copy(
                    vy2.at[p, j], acc_y.at[row_idx], sem, add=True))
                descs.append(pltpu.async_copy(
                    vz2.at[p, j], acc_z.at[row_idx], sem, add=True))
            return descs

        st = stage(0)
        prev = None
        for b in range(n_batches):
            for d in st:
                d.wait()
            cur = fire(b)
            if prev is not None:
                for d in prev:
                    d.wait()
            if b + 1 < n_batches:
                st = stage(b + 1)
            prev = cur
        for d in prev:
            d.wait()
        plsc.subcore_barrier()
        pltpu.sync_copy(acc_x.at[sl], out_hbm.at[c * 3 + 0, sl])
        pltpu.sync_copy(acc_y.at[sl], out_hbm.at[c * 3 + 1, sl])
        pltpu.sync_copy(acc_z.at[sl], out_hbm.at[c * 3 + 2, sl])

    return sc_scatter


def _combine_body(a_ref, b_ref, out_ref):
    out_ref[...] = a_ref[...] + b_ref[...]


def _combine_call(a, b):
    return pl.pallas_call(
        _combine_body,
        out_shape=jax.ShapeDtypeStruct(a.shape, jnp.float32),
    )(a, b)


def kernel(force_features, edge_vectors, pos, edge_index_dst, W0, b0, W1, b1):
    e, d = force_features.shape
    n = pos.shape[0]
    n_pad = ((n + 255) // 256) * 256
    rows = e // 128
    rows_pad = ((rows + 32 * _K - 1) // (32 * _K)) * (32 * _K)

    grid = e // (128 * _BR)
    ev3 = edge_vectors.reshape(grid, _BR, 384)
    lane = jnp.arange(128)
    comp = jnp.arange(3)
    p_rows = (3 * lane[None, :] + comp[:, None]).ravel()
    p_cols = (128 * comp[:, None] + lane[None, :]).ravel()
    pmat = jnp.zeros((384, 384), jnp.float32).at[p_rows, p_cols].set(1.0)
    b0r = b0.reshape(1, d)
    b1r = b1.reshape(1, 1)

    vx, vy, vz = _mlp_call(force_features, ev3, pmat, W0, b0r, W1, b1r)
    vx = vx.reshape(rows, 128)
    vy = vy.reshape(rows, 128)
    vz = vz.reshape(rows, 128)

    pad_rows = rows_pad - rows
    padv = ((0, pad_rows), (0, 0))
    vx = jnp.pad(vx, padv)
    vy = jnp.pad(vy, padv)
    vz = jnp.pad(vz, padv)
    pad_idx = (jnp.arange(pad_rows * 128, dtype=jnp.int32) % n).reshape(
        pad_rows, 128)
    dst2d = jnp.concatenate(
        [edge_index_dst.reshape(rows, 128), pad_idx], axis=0)

    zeros = jnp.zeros((n_pad,), dtype=jnp.float32)
    partials = _make_sc_scatter(rows_pad, n_pad)(vx, vy, vz, dst2d, zeros)

    out = _combine_call(partials[0:3], partials[3:6])
    return out.T[:n, :]
